# Initial kernel scaffold; baseline (speedup 1.0000x reference)
import jax
import jax.numpy as jnp
from jax import lax
from jax.experimental import pallas as pl
from jax.experimental.pallas import tpu as pltpu
from jax.experimental.pallas import tpu_sc as plsc

VOC = 1000000
D = 64
B = 16384
S = 200

NC = 2
NS = 16
NW = NC * NS
BPW = B // NW
RB = 64
NRB = BPW // RB
G0 = 128
G1 = S - G0

_LANES = 16
_NACC = D // _LANES


def _cbow_kernel(x_hbm, table_hbm, out_hbm, idxs, rows, outstage, sem):
    wid = lax.axis_index("s") * NC + lax.axis_index("c")
    row0 = wid * BPW

    def block(g, carry):
        base = row0 + g * RB
        pltpu.sync_copy(x_hbm.at[pl.ds(base, RB), :], idxs)

        def per_row(r, c2):
            cp0 = pltpu.async_copy(
                table_hbm.at[idxs.at[r, pl.ds(0, G0)]],
                rows.at[pl.ds(0, G0), :], sem)
            cp1 = pltpu.async_copy(
                table_hbm.at[idxs.at[r, pl.ds(G0, G1)]],
                rows.at[pl.ds(G0, G1), :], sem)
            cp0.wait()
            cp1.wait()

            def red(s2, accs):
                return tuple(
                    accs[c] + rows[s2, pl.ds(c * _LANES, _LANES)]
                    for c in range(_NACC))

            zero = jnp.zeros((_LANES,), jnp.float32)
            accs = lax.fori_loop(0, S, red, (zero,) * _NACC)
            scale = jnp.float32(1.0 / S)
            for c in range(_NACC):
                outstage[r, pl.ds(c * _LANES, _LANES)] = accs[c] * scale
            return c2

        lax.fori_loop(0, RB, per_row, 0)
        pltpu.sync_copy(outstage, out_hbm.at[pl.ds(base, RB), :])
        return carry

    lax.fori_loop(0, NRB, block, 0)


@jax.jit
def _cbow(x, table):
    mesh = plsc.VectorSubcoreMesh(
        core_axis_name="c", subcore_axis_name="s",
        num_cores=NC, num_subcores=NS)
    run = pl.kernel(
        _cbow_kernel,
        out_type=jax.ShapeDtypeStruct((B, D), jnp.float32),
        mesh=mesh,
        scratch_types=[
            pltpu.VMEM((RB, S), jnp.int32),
            pltpu.VMEM((S, D), jnp.float32),
            pltpu.VMEM((RB, D), jnp.float32),
            pltpu.SemaphoreType.DMA,
        ],
    )
    return run(x, table)


def kernel(x, word_pos, x_char, unused, table):
    del word_pos, x_char, unused
    return _cbow(x.astype(jnp.int32), table)


# speedup vs baseline: 2.0157x; 2.0157x over previous
import jax
import jax.numpy as jnp
from jax import lax
from jax.experimental import pallas as pl
from jax.experimental.pallas import tpu as pltpu
from jax.experimental.pallas import tpu_sc as plsc

VOC = 1000000
D = 64
B = 16384
S = 200

NC = 2
NS = 16
NW = NC * NS
BPW = B // NW
RB = 64
NRB = BPW // RB
G0 = 128
G1 = S - G0

_LANES = 16
_NACC = D // _LANES


def _cbow_kernel(x_hbm, table_hbm, out_hbm, idxs, rows, outstage, sem):
    wid = lax.axis_index("s") * NC + lax.axis_index("c")
    row0 = wid * BPW

    def block(g, carry):
        base = row0 + g * RB
        pltpu.sync_copy(x_hbm.at[pl.ds(base, RB), :], idxs)

        def per_row(r, c2):
            cp0 = pltpu.async_copy(
                table_hbm.at[idxs.at[r, pl.ds(0, G0)]],
                rows.at[pl.ds(0, G0), :], sem)
            cp1 = pltpu.async_copy(
                table_hbm.at[idxs.at[r, pl.ds(G0, G1)]],
                rows.at[pl.ds(G0, G1), :], sem)
            cp0.wait()
            cp1.wait()

            def red(s2, accs):
                return tuple(
                    accs[c] + rows[s2, pl.ds(c * _LANES, _LANES)]
                    for c in range(_NACC))

            zero = jnp.zeros((_LANES,), jnp.float32)
            accs = lax.fori_loop(0, S, red, (zero,) * _NACC)
            scale = jnp.float32(1.0 / S)
            for c in range(_NACC):
                outstage[r, pl.ds(c * _LANES, _LANES)] = accs[c] * scale
            return c2

        lax.fori_loop(0, RB, per_row, 0)
        pltpu.sync_copy(outstage, out_hbm.at[pl.ds(base, RB), :])
        return carry

    lax.fori_loop(0, NRB, block, 0)


@jax.jit
def _cbow(x, table):
    mesh = plsc.VectorSubcoreMesh(
        core_axis_name="c", subcore_axis_name="s",
        num_cores=NC, num_subcores=NS)
    run = pl.kernel(
        _cbow_kernel,
        out_type=jax.ShapeDtypeStruct((B, D), jnp.float32),
        mesh=mesh,
        scratch_types=[
            pltpu.VMEM((RB, S), jnp.int32),
            pltpu.VMEM((S, D), jnp.float32),
            pltpu.VMEM((RB, D), jnp.float32),
            pltpu.SemaphoreType.DMA,
        ],
        compiler_params=pltpu.CompilerParams(use_tc_tiling_on_sc=False),
    )
    return run(x, table)


def kernel(x, word_pos, x_char, unused, table):
    del word_pos, x_char, unused
    return _cbow(x.astype(jnp.int32), table)


# speedup vs baseline: 3.4638x; 1.7184x over previous
import jax
import jax.numpy as jnp
from jax import lax
from jax.experimental import pallas as pl
from jax.experimental.pallas import tpu as pltpu
from jax.experimental.pallas import tpu_sc as plsc

VOC = 1000000
D = 64
B = 16384
S = 200

NC = 2
NS = 16
NW = NC * NS
BPW = B // NW
RB = 128
NRB = BPW // RB
G0 = 128
G1 = S - G0
NBUF = 4

_LANES = 16
_NACC = D // _LANES


def _cbow_kernel(x_hbm, table_hbm, out_hbm, idxs, rows, outstage, sems):
    wid = lax.axis_index("s") * NC + lax.axis_index("c")
    row0 = wid * BPW
    scale = jnp.float32(1.0 / S)

    def start(r, b):
        pltpu.async_copy(
            table_hbm.at[idxs.at[r, pl.ds(0, G0)]],
            rows.at[b, pl.ds(0, G0), :], sems.at[b])
        pltpu.async_copy(
            table_hbm.at[idxs.at[r, pl.ds(G0, G1)]],
            rows.at[b, pl.ds(G0, G1), :], sems.at[b])

    def wait(b):
        pltpu.make_async_copy(
            table_hbm.at[idxs.at[0, pl.ds(0, G0)]],
            rows.at[b, pl.ds(0, G0), :], sems.at[b]).wait()
        pltpu.make_async_copy(
            table_hbm.at[idxs.at[0, pl.ds(G0, G1)]],
            rows.at[b, pl.ds(G0, G1), :], sems.at[b]).wait()

    def reduce_into(r, b):
        def red(s2, accs):
            return tuple(
                accs[c] + rows[b, s2, pl.ds(c * _LANES, _LANES)]
                for c in range(_NACC))

        zero = jnp.zeros((_LANES,), jnp.float32)
        accs = lax.fori_loop(0, S, red, (zero,) * _NACC, unroll=4)
        for c in range(_NACC):
            outstage[r, pl.ds(c * _LANES, _LANES)] = accs[c] * scale

    def block(g, carry):
        base = row0 + g * RB
        pltpu.sync_copy(x_hbm.at[pl.ds(base, RB), :], idxs)
        for b in range(NBUF):
            start(b, b)

        def group(p, c2):
            for b in range(NBUF):
                r = p * NBUF + b
                wait(b)
                reduce_into(r, b)
                nxt = r + NBUF

                @pl.when(nxt < RB)
                def _():
                    start(nxt, b)
            return c2

        lax.fori_loop(0, RB // NBUF, group, 0)
        pltpu.sync_copy(outstage, out_hbm.at[pl.ds(base, RB), :])
        return carry

    lax.fori_loop(0, NRB, block, 0)


@jax.jit
def _cbow(x, table):
    mesh = plsc.VectorSubcoreMesh(
        core_axis_name="c", subcore_axis_name="s",
        num_cores=NC, num_subcores=NS)
    run = pl.kernel(
        _cbow_kernel,
        out_type=jax.ShapeDtypeStruct((B, D), jnp.float32),
        mesh=mesh,
        scratch_types=[
            pltpu.VMEM((RB, S), jnp.int32),
            pltpu.VMEM((NBUF, S, D), jnp.float32),
            pltpu.VMEM((RB, D), jnp.float32),
            pltpu.SemaphoreType.DMA((NBUF,)),
        ],
        compiler_params=pltpu.CompilerParams(use_tc_tiling_on_sc=False),
    )
    return run(x, table)


def kernel(x, word_pos, x_char, unused, table):
    del word_pos, x_char, unused
    return _cbow(x.astype(jnp.int32), table)


# speedup vs baseline: 3.5005x; 1.0106x over previous
import jax
import jax.numpy as jnp
from jax import lax
from jax.experimental import pallas as pl
from jax.experimental.pallas import tpu as pltpu
from jax.experimental.pallas import tpu_sc as plsc

VOC = 1000000
D = 64
B = 16384
S = 200

NC = 2
NS = 16
NW = NC * NS
BPW = B // NW
C = 128
NCH = BPW // C

_LANES = 16
_NACC = D // _LANES


def _cbow_kernel(xT_hbm, table_hbm, out_hbm, idxT, acc, sem):
    wid = lax.axis_index("s") * NC + lax.axis_index("c")
    row0 = wid * BPW
    scale = jnp.float32(1.0 / S)

    def chunk(g, carry):
        base = row0 + g * C
        pltpu.sync_copy(xT_hbm.at[:, pl.ds(base, C)], idxT)

        def z(i, c2):
            for c in range(_NACC):
                acc[i, pl.ds(c * _LANES, _LANES)] = jnp.zeros((_LANES,), jnp.float32)
            return c2
        lax.fori_loop(0, C, z, 0)

        def fire(s2, c2):
            pltpu.async_copy(table_hbm.at[idxT.at[s2]], acc, sem, add=True)
            return c2
        lax.fori_loop(0, S, fire, 0)

        def drain(s2, c2):
            pltpu.make_async_copy(table_hbm.at[idxT.at[0]], acc, sem).wait()
            return c2
        lax.fori_loop(0, S, drain, 0)

        def sc(i, c2):
            for c in range(_NACC):
                sl = pl.ds(c * _LANES, _LANES)
                acc[i, sl] = acc[i, sl] * scale
            return c2
        lax.fori_loop(0, C, sc, 0)
        pltpu.sync_copy(acc, out_hbm.at[pl.ds(base, C), :])
        return carry

    lax.fori_loop(0, NCH, chunk, 0)


@jax.jit
def _cbow(xT, table):
    mesh = plsc.VectorSubcoreMesh(
        core_axis_name="c", subcore_axis_name="s",
        num_cores=NC, num_subcores=NS)
    run = pl.kernel(
        _cbow_kernel,
        out_type=jax.ShapeDtypeStruct((B, D), jnp.float32),
        mesh=mesh,
        scratch_types=[
            pltpu.VMEM((S, C), jnp.int32),
            pltpu.VMEM((C, D), jnp.float32),
            pltpu.SemaphoreType.DMA,
        ],
        compiler_params=pltpu.CompilerParams(use_tc_tiling_on_sc=False),
    )
    return run(xT, table)


def kernel(x, word_pos, x_char, unused, table):
    del word_pos, x_char, unused
    return _cbow(x.astype(jnp.int32).T, table)
